# baseline (speedup 1.0000x reference)
import functools

import jax
import jax.numpy as jnp
from jax import lax
from jax.experimental import pallas as pl
from jax.experimental.pallas import tpu as pltpu
from jax.experimental.pallas import tpu_sc as plsc

_WIDTH = 1024
_TOTAL_WIDTH = 100000
_N = 2097152

_NC = 2
_NS = 16
_NW = _NC * _NS
_PER_W = _N // _NW
_CHUNK = 8192
_TBL = 1040
_LN2 = 0.6931471805599453
_LOG_OUT_DENOM = 11.502619817772691


def _vlog(x):
    bits = plsc.bitcast(x, jnp.int32)
    e = (bits >> 23) - 127
    m = plsc.bitcast((bits & 0x007FFFFF) | 0x3F800000, jnp.float32)
    u = (m - 1.0) / (m + 1.0)
    p = u * u
    poly = 1.0 + p * (1.0 / 3.0 + p * (0.2 + p * (1.0 / 7.0 + p * (1.0 / 9.0))))
    return e.astype(jnp.float32) * _LN2 + 2.0 * u * poly


def _sc_body(c0_ref, c1_ref, h0_ref, h1_ref, h2_ref, h3_ref, lpi_ref, z_ref,
             out_ref, a0, a1, b0, b1, o0, o1, t0, t1, t2, t3, tz, tlpi,
             lz1, lz2, lz3, tbl,
             sa0, sa1, sb0, sb1, so0, so1):
    wid = lax.axis_index("s") * _NC + lax.axis_index("c")
    abufs = (a0, a1)
    bbufs = (b0, b1)
    obufs = (o0, o1)
    sas = (sa0, sa1)
    sbs = (sb0, sb1)
    sos = (so0, so1)
    nchunks = _PER_W // _CHUNK
    in_handles = [None, None]
    out_handles = [None, None]

    def start_in(chunk):
        sl = chunk % 2
        base = wid * _PER_W + chunk * _CHUNK
        ha = pltpu.async_copy(c0_ref.at[pl.ds(base, _CHUNK)], abufs[sl], sas[sl])
        hb = pltpu.async_copy(c1_ref.at[pl.ds(base, _CHUNK)], bbufs[sl], sbs[sl])
        in_handles[sl] = (ha, hb)

    start_in(0)
    pltpu.sync_copy(h0_ref, t0)
    pltpu.sync_copy(h1_ref, t1)
    pltpu.sync_copy(h2_ref, t2)
    pltpu.sync_copy(h3_ref, t3)
    pltpu.sync_copy(z_ref, tz)
    pltpu.sync_copy(lpi_ref, tlpi)
    lanes = lax.iota(jnp.int32, 16)
    zv = tz[...]

    def row_logz(tab, rows, ncols):
        cols = [plsc.load_gather(tab, [rows, zv + j]) for j in range(ncols)]
        m = cols[0]
        for v in cols[1:]:
            m = jnp.maximum(m, v)
        se = jnp.exp(cols[0] - m)
        for v in cols[1:]:
            se = se + jnp.exp(v - m)
        return m + _vlog(se)

    lz0v = row_logz(t0, zv, 8)
    lz1[pl.ds(0, 16)] = row_logz(t1, lanes & 7, 8)

    @plsc.parallel_loop(0, 4)
    def build_lz2(k):
        rows = lanes + k * 16
        lz2[pl.ds(k * 16, 16)] = row_logz(t2, rows, 8)

    @plsc.parallel_loop(0, 32, unroll=4)
    def build_lz3(k):
        rows = lanes + k * 16
        lz3[pl.ds(k * 16, 16)] = row_logz(t3, rows, 2)

    x = tlpi[...]
    pin = 1.0 / (1.0 + jnp.exp(-x))
    lpi = _vlog(pin)
    lpo = _vlog(1.0 - pin) - _LOG_OUT_DENOM

    @plsc.parallel_loop(0, _TBL // 16, unroll=5)
    def build(k):
        s = lanes + k * 16
        s_c = jnp.minimum(s, _WIDTH - 1)
        v = (lpi
             + plsc.load_gather(t0, [zv, s_c >> 7]) - lz0v
             + plsc.load_gather(t1, [s_c >> 7, (s_c >> 4) & 7])
             - plsc.load_gather(lz1, [s_c >> 7])
             + plsc.load_gather(t2, [s_c >> 4, (s_c >> 1) & 7])
             - plsc.load_gather(lz2, [s_c >> 4])
             + plsc.load_gather(t3, [s_c >> 1, s_c & 1])
             - plsc.load_gather(lz3, [s_c >> 1]))
        tbl[pl.ds(k * 16, 16)] = jnp.where(s < _WIDTH, v, lpo)

    for chunk in range(nchunks):
        sl = chunk % 2
        if chunk + 1 < nchunks:
            start_in(chunk + 1)
        ha, hb = in_handles[sl]
        ha.wait()
        hb.wait()
        if out_handles[sl] is not None:
            out_handles[sl].wait()
        avmem, bvmem, ovmem = abufs[sl], bbufs[sl], obufs[sl]

        @plsc.parallel_loop(0, _CHUNK // 16, unroll=8)
        def body(i):
            c0 = avmem[pl.ds(i * 16, 16)]
            c1 = bvmem[pl.ds(i * 16, 16)]
            idx = jnp.minimum(jnp.abs(c1 - c0), _WIDTH)
            ovmem[pl.ds(i * 16, 16)] = plsc.load_gather(tbl, [idx])

        base = wid * _PER_W + chunk * _CHUNK
        out_handles[sl] = pltpu.async_copy(
            ovmem, out_ref.at[pl.ds(base, _CHUNK)], sos[sl])

    for sl in (0, 1):
        if out_handles[sl] is not None:
            out_handles[sl].wait()


@functools.cache
def _sc_lookup():
    return pl.kernel(
        _sc_body,
        out_type=jax.ShapeDtypeStruct((_N,), jnp.float32),
        mesh=plsc.VectorSubcoreMesh(
            core_axis_name="c", subcore_axis_name="s",
            num_cores=_NC, num_subcores=_NS),
        scratch_types=[
            pltpu.VMEM((_CHUNK,), jnp.int32),
            pltpu.VMEM((_CHUNK,), jnp.int32),
            pltpu.VMEM((_CHUNK,), jnp.int32),
            pltpu.VMEM((_CHUNK,), jnp.int32),
            pltpu.VMEM((_CHUNK,), jnp.float32),
            pltpu.VMEM((_CHUNK,), jnp.float32),
            pltpu.VMEM((1, 8), jnp.float32),
            pltpu.VMEM((8, 8), jnp.float32),
            pltpu.VMEM((64, 8), jnp.float32),
            pltpu.VMEM((512, 2), jnp.float32),
            pltpu.VMEM((16,), jnp.int32),
            pltpu.VMEM((16,), jnp.float32),
            pltpu.VMEM((16,), jnp.float32),
            pltpu.VMEM((64,), jnp.float32),
            pltpu.VMEM((512,), jnp.float32),
            pltpu.VMEM((_TBL,), jnp.float32),
            pltpu.SemaphoreType.DMA,
            pltpu.SemaphoreType.DMA,
            pltpu.SemaphoreType.DMA,
            pltpu.SemaphoreType.DMA,
            pltpu.SemaphoreType.DMA,
            pltpu.SemaphoreType.DMA,
        ],
        compiler_params=pltpu.CompilerParams(
            needs_layout_passes=False, use_tc_tiling_on_sc=False),
    )


def kernel(coordinates, h0, h1, h2, h3, logprob_inside):
    lpiv = jnp.broadcast_to(logprob_inside.reshape(1), (16,)).astype(jnp.float32)
    return _sc_lookup()(
        coordinates[:, 0], coordinates[:, 1],
        h0, h1, h2, h3, lpiv, jnp.zeros((16,), jnp.int32))


# speedup vs baseline: 1.0229x; 1.0229x over previous
import functools

import jax
import jax.numpy as jnp
from jax import lax
from jax.experimental import pallas as pl
from jax.experimental.pallas import tpu as pltpu
from jax.experimental.pallas import tpu_sc as plsc

_WIDTH = 1024
_TOTAL_WIDTH = 100000
_N = 2097152

_NC = 2
_NS = 16
_NW = _NC * _NS
_PER_W = _N // _NW
_CHUNK = 16384
_TBL = 1040
_LN2 = 0.6931471805599453
_LOG_OUT_DENOM = 11.502619817772691


def _vlog(x):
    bits = plsc.bitcast(x, jnp.int32)
    e = (bits >> 23) - 127
    m = plsc.bitcast((bits & 0x007FFFFF) | 0x3F800000, jnp.float32)
    u = (m - 1.0) / (m + 1.0)
    p = u * u
    poly = 1.0 + p * (1.0 / 3.0 + p * (0.2 + p * (1.0 / 7.0 + p * (1.0 / 9.0))))
    return e.astype(jnp.float32) * _LN2 + 2.0 * u * poly


def _sc_body(c0_ref, c1_ref, h0_ref, h1_ref, h2_ref, h3_ref, lpi_ref, z_ref,
             out_ref, a0, a1, b0, b1, o0, o1, t0, t1, t2, t3, tz, tlpi,
             lz1, lz2, lz3, tbl,
             sa0, sa1, sb0, sb1, so0, so1):
    wid = lax.axis_index("s") * _NC + lax.axis_index("c")
    abufs = (a0, a1)
    bbufs = (b0, b1)
    obufs = (o0, o1)
    sas = (sa0, sa1)
    sbs = (sb0, sb1)
    sos = (so0, so1)
    nchunks = _PER_W // _CHUNK
    in_handles = [None, None]
    out_handles = [None, None]

    def start_in(chunk):
        sl = chunk % 2
        base = wid * _PER_W + chunk * _CHUNK
        ha = pltpu.async_copy(c0_ref.at[pl.ds(base, _CHUNK)], abufs[sl], sas[sl])
        hb = pltpu.async_copy(c1_ref.at[pl.ds(base, _CHUNK)], bbufs[sl], sbs[sl])
        in_handles[sl] = (ha, hb)

    start_in(0)
    pltpu.sync_copy(h0_ref, t0)
    pltpu.sync_copy(h1_ref, t1)
    pltpu.sync_copy(h2_ref, t2)
    pltpu.sync_copy(h3_ref, t3)
    pltpu.sync_copy(z_ref, tz)
    pltpu.sync_copy(lpi_ref, tlpi)
    lanes = lax.iota(jnp.int32, 16)
    zv = tz[...]

    def row_logz(tab, rows, ncols):
        cols = [plsc.load_gather(tab, [rows, zv + j]) for j in range(ncols)]
        m = cols[0]
        for v in cols[1:]:
            m = jnp.maximum(m, v)
        se = jnp.exp(cols[0] - m)
        for v in cols[1:]:
            se = se + jnp.exp(v - m)
        return m + _vlog(se)

    lz0v = row_logz(t0, zv, 8)
    lz1[pl.ds(0, 16)] = row_logz(t1, lanes & 7, 8)

    @plsc.parallel_loop(0, 4)
    def build_lz2(k):
        rows = lanes + k * 16
        lz2[pl.ds(k * 16, 16)] = row_logz(t2, rows, 8)

    @plsc.parallel_loop(0, 32, unroll=4)
    def build_lz3(k):
        rows = lanes + k * 16
        lz3[pl.ds(k * 16, 16)] = row_logz(t3, rows, 2)

    x = plsc.load_gather(tlpi, [zv])
    pin = 1.0 / (1.0 + jnp.exp(-x))
    lpi = _vlog(pin)
    lpo = _vlog(1.0 - pin) - _LOG_OUT_DENOM

    @plsc.parallel_loop(0, _TBL // 16, unroll=5)
    def build(k):
        s = lanes + k * 16
        s_c = jnp.minimum(s, _WIDTH - 1)
        v = (lpi
             + plsc.load_gather(t0, [zv, s_c >> 7]) - lz0v
             + plsc.load_gather(t1, [s_c >> 7, (s_c >> 4) & 7])
             - plsc.load_gather(lz1, [s_c >> 7])
             + plsc.load_gather(t2, [s_c >> 4, (s_c >> 1) & 7])
             - plsc.load_gather(lz2, [s_c >> 4])
             + plsc.load_gather(t3, [s_c >> 1, s_c & 1])
             - plsc.load_gather(lz3, [s_c >> 1]))
        tbl[pl.ds(k * 16, 16)] = jnp.where(s < _WIDTH, v, lpo)

    for chunk in range(nchunks):
        sl = chunk % 2
        if chunk + 1 < nchunks:
            start_in(chunk + 1)
        ha, hb = in_handles[sl]
        ha.wait()
        hb.wait()
        if out_handles[sl] is not None:
            out_handles[sl].wait()
        avmem, bvmem, ovmem = abufs[sl], bbufs[sl], obufs[sl]

        @plsc.parallel_loop(0, _CHUNK // 16, unroll=8)
        def body(i):
            c0 = avmem[pl.ds(i * 16, 16)]
            c1 = bvmem[pl.ds(i * 16, 16)]
            idx = jnp.minimum(jnp.abs(c1 - c0), _WIDTH)
            ovmem[pl.ds(i * 16, 16)] = plsc.load_gather(tbl, [idx])

        base = wid * _PER_W + chunk * _CHUNK
        out_handles[sl] = pltpu.async_copy(
            ovmem, out_ref.at[pl.ds(base, _CHUNK)], sos[sl])

    for sl in (0, 1):
        if out_handles[sl] is not None:
            out_handles[sl].wait()


@functools.cache
def _sc_lookup():
    return pl.kernel(
        _sc_body,
        out_type=jax.ShapeDtypeStruct((_N,), jnp.float32),
        mesh=plsc.VectorSubcoreMesh(
            core_axis_name="c", subcore_axis_name="s",
            num_cores=_NC, num_subcores=_NS),
        scratch_types=[
            pltpu.VMEM((_CHUNK,), jnp.int32),
            pltpu.VMEM((_CHUNK,), jnp.int32),
            pltpu.VMEM((_CHUNK,), jnp.int32),
            pltpu.VMEM((_CHUNK,), jnp.int32),
            pltpu.VMEM((_CHUNK,), jnp.float32),
            pltpu.VMEM((_CHUNK,), jnp.float32),
            pltpu.VMEM((1, 8), jnp.float32),
            pltpu.VMEM((8, 8), jnp.float32),
            pltpu.VMEM((64, 8), jnp.float32),
            pltpu.VMEM((512, 2), jnp.float32),
            pltpu.VMEM((16,), jnp.int32),
            pltpu.VMEM((1,), jnp.float32),
            pltpu.VMEM((16,), jnp.float32),
            pltpu.VMEM((64,), jnp.float32),
            pltpu.VMEM((512,), jnp.float32),
            pltpu.VMEM((_TBL,), jnp.float32),
            pltpu.SemaphoreType.DMA,
            pltpu.SemaphoreType.DMA,
            pltpu.SemaphoreType.DMA,
            pltpu.SemaphoreType.DMA,
            pltpu.SemaphoreType.DMA,
            pltpu.SemaphoreType.DMA,
        ],
        compiler_params=pltpu.CompilerParams(
            needs_layout_passes=False, use_tc_tiling_on_sc=False),
    )


def kernel(coordinates, h0, h1, h2, h3, logprob_inside):
    return _sc_lookup()(
        coordinates[:, 0], coordinates[:, 1],
        h0, h1, h2, h3, logprob_inside.reshape(1), jnp.zeros((16,), jnp.int32))


# speedup vs baseline: 1.0244x; 1.0015x over previous
import functools

import jax
import jax.numpy as jnp
from jax import lax
from jax.experimental import pallas as pl
from jax.experimental.pallas import tpu as pltpu
from jax.experimental.pallas import tpu_sc as plsc

_WIDTH = 1024
_TOTAL_WIDTH = 100000
_N = 2097152

_NC = 2
_NS = 16
_NW = _NC * _NS
_PER_W = _N // _NW
_CHUNK = 16384
_TBL = 1040
_LN2 = 0.6931471805599453
_LOG_OUT_DENOM = 11.502619817772691


def _vlog(x):
    bits = plsc.bitcast(x, jnp.int32)
    e = (bits >> 23) - 127
    m = plsc.bitcast((bits & 0x007FFFFF) | 0x3F800000, jnp.float32)
    u = (m - 1.0) / (m + 1.0)
    p = u * u
    poly = 1.0 + p * (1.0 / 3.0 + p * (0.2 + p * (1.0 / 7.0 + p * (1.0 / 9.0))))
    return e.astype(jnp.float32) * _LN2 + 2.0 * u * poly


def _sc_body(c0_ref, c1_ref, h0_ref, h1_ref, h2_ref, h3_ref, lpi_ref, z_ref,
             out_ref, a0, a1, b0, b1, o0, o1, t0, t1, t2, t3, tz, tlpi,
             lz1, lz2, lz3, tbl,
             sa0, sa1, sb0, sb1, so0, so1):
    wid = lax.axis_index("s") * _NC + lax.axis_index("c")
    abufs = (a0, a1)
    bbufs = (b0, b1)
    obufs = (o0, o1)
    sas = (sa0, sa1)
    sbs = (sb0, sb1)
    sos = (so0, so1)
    nchunks = _PER_W // _CHUNK
    in_handles = [None, None]
    out_handles = [None, None]

    def start_in(chunk):
        sl = chunk % 2
        base = wid * _PER_W + chunk * _CHUNK
        ha = pltpu.async_copy(c0_ref.at[pl.ds(base, _CHUNK)], abufs[sl], sas[sl])
        hb = pltpu.async_copy(c1_ref.at[pl.ds(base, _CHUNK)], bbufs[sl], sbs[sl])
        in_handles[sl] = (ha, hb)

    start_in(0)
    pltpu.sync_copy(h0_ref, t0)
    pltpu.sync_copy(h1_ref, t1)
    pltpu.sync_copy(h2_ref, t2)
    pltpu.sync_copy(h3_ref, t3)
    pltpu.sync_copy(z_ref, tz)
    pltpu.sync_copy(lpi_ref, tlpi)
    lanes = lax.iota(jnp.int32, 16)
    zv = tz[...]

    def row_logz(tab, rows, ncols):
        cols = [plsc.load_gather(tab, [rows, zv + j]) for j in range(ncols)]
        m = cols[0]
        for v in cols[1:]:
            m = jnp.maximum(m, v)
        se = jnp.exp(cols[0] - m)
        for v in cols[1:]:
            se = se + jnp.exp(v - m)
        return m + _vlog(se)

    lz0v = row_logz(t0, zv, 8)
    lz1[pl.ds(0, 16)] = row_logz(t1, lanes & 7, 8)

    @plsc.parallel_loop(0, 4)
    def build_lz2(k):
        rows = lanes + k * 16
        lz2[pl.ds(k * 16, 16)] = row_logz(t2, rows, 8)

    @plsc.parallel_loop(0, 32, unroll=4)
    def build_lz3(k):
        rows = lanes + k * 16
        lz3[pl.ds(k * 16, 16)] = row_logz(t3, rows, 2)

    x = plsc.load_gather(tlpi, [zv])
    pin = 1.0 / (1.0 + jnp.exp(-x))
    lpi = _vlog(pin)
    lpo = _vlog(1.0 - pin) - _LOG_OUT_DENOM

    @plsc.parallel_loop(0, _TBL // 16, unroll=5)
    def build(k):
        s = lanes + k * 16
        s_c = jnp.minimum(s, _WIDTH - 1)
        v = (lpi
             + plsc.load_gather(t0, [zv, s_c >> 7]) - lz0v
             + plsc.load_gather(t1, [s_c >> 7, (s_c >> 4) & 7])
             - plsc.load_gather(lz1, [s_c >> 7])
             + plsc.load_gather(t2, [s_c >> 4, (s_c >> 1) & 7])
             - plsc.load_gather(lz2, [s_c >> 4])
             + plsc.load_gather(t3, [s_c >> 1, s_c & 1])
             - plsc.load_gather(lz3, [s_c >> 1]))
        tbl[pl.ds(k * 16, 16)] = jnp.where(s < _WIDTH, v, lpo)

    for chunk in range(nchunks):
        sl = chunk % 2
        if chunk + 1 < nchunks:
            start_in(chunk + 1)
        ha, hb = in_handles[sl]
        ha.wait()
        hb.wait()
        if out_handles[sl] is not None:
            out_handles[sl].wait()
        avmem, bvmem, ovmem = abufs[sl], bbufs[sl], obufs[sl]

        @plsc.parallel_loop(0, _CHUNK // 16, unroll=8)
        def body(i):
            c0 = avmem[pl.ds(i * 16, 16)]
            c1 = bvmem[pl.ds(i * 16, 16)]
            idx = jnp.minimum(jnp.abs(c1 - c0), _WIDTH)
            ovmem[pl.ds(i * 16, 16)] = plsc.load_gather(tbl, [idx])

        base = wid * _PER_W + chunk * _CHUNK
        out_handles[sl] = pltpu.async_copy(
            ovmem, out_ref.at[pl.ds(base, _CHUNK)], sos[sl])

    for sl in (0, 1):
        if out_handles[sl] is not None:
            out_handles[sl].wait()


@functools.cache
def _sc_lookup():
    return pl.kernel(
        _sc_body,
        out_type=jax.ShapeDtypeStruct((_N,), jnp.float32),
        mesh=plsc.VectorSubcoreMesh(
            core_axis_name="c", subcore_axis_name="s",
            num_cores=_NC, num_subcores=_NS),
        scratch_types=[
            pltpu.VMEM((_CHUNK,), jnp.int32),
            pltpu.VMEM((_CHUNK,), jnp.int32),
            pltpu.VMEM((_CHUNK,), jnp.int32),
            pltpu.VMEM((_CHUNK,), jnp.int32),
            pltpu.VMEM((_CHUNK,), jnp.float32),
            pltpu.VMEM((_CHUNK,), jnp.float32),
            pltpu.VMEM((1, 8), jnp.float32),
            pltpu.VMEM((8, 8), jnp.float32),
            pltpu.VMEM((64, 8), jnp.float32),
            pltpu.VMEM((512, 2), jnp.float32),
            pltpu.VMEM((16,), jnp.int32),
            pltpu.VMEM((1,), jnp.float32),
            pltpu.VMEM((16,), jnp.float32),
            pltpu.VMEM((64,), jnp.float32),
            pltpu.VMEM((512,), jnp.float32),
            pltpu.VMEM((_TBL,), jnp.float32),
            pltpu.SemaphoreType.DMA,
            pltpu.SemaphoreType.DMA,
            pltpu.SemaphoreType.DMA,
            pltpu.SemaphoreType.DMA,
            pltpu.SemaphoreType.DMA,
            pltpu.SemaphoreType.DMA,
        ],
        compiler_params=pltpu.CompilerParams(
            needs_layout_passes=False, use_tc_tiling_on_sc=False),
    )


def kernel(coordinates, h0, h1, h2, h3, logprob_inside):
    ct = coordinates.T
    return _sc_lookup()(
        ct[0], ct[1],
        h0, h1, h2, h3, logprob_inside.reshape(1), jnp.zeros((16,), jnp.int32))
